# baseline (speedup 1.0000x reference)
import functools

import jax
import jax.numpy as jnp
from jax import lax
from jax.experimental import pallas as pl

_NE = 64
_K = 8
_EPS = 1e-6


def _gate_kernel(x_ref, w_ref, b_ref, out_ref):
    x = x_ref[...]
    w = w_ref[...]
    logits = lax.dot_general(x, w, (((1,), (1,)), ((), ())),
                             preferred_element_type=jnp.float32)
    logits = logits + b_ref[...]
    t = logits.shape[0]
    idx = lax.broadcasted_iota(jnp.int32, (t, _NE), 1)
    m = jnp.max(logits, axis=1, keepdims=True)
    e = jnp.exp(logits - m)
    z = jnp.sum(e, axis=1, keepdims=True)
    tv = jnp.full((t, 1), jnp.inf, jnp.float32)
    ti = jnp.full((t, 1), -1, jnp.int32)
    for _ in range(_K):
        elig = (logits < tv) | ((logits == tv) & (idx > ti))
        lm = jnp.where(elig, logits, -jnp.inf)
        tv = jnp.max(lm, axis=1, keepdims=True)
        ti = jnp.min(jnp.where(lm == tv, idx, _NE), axis=1, keepdims=True)
    mask = (logits > tv) | ((logits == tv) & (idx <= ti))
    es = jnp.where(mask, e, 0.0)
    s8 = jnp.sum(es, axis=1, keepdims=True)
    out_ref[...] = es / (s8 + _EPS * z)


@functools.partial(jax.jit, static_argnames=("block_t",))
def _switch_gate(x, w, b, block_t=1024):
    bsz, seq, d = x.shape
    n_tok = bsz * seq
    xf = x.reshape(n_tok, d)
    grid = n_tok // block_t
    out = pl.pallas_call(
        _gate_kernel,
        grid=(grid,),
        in_specs=[
            pl.BlockSpec((block_t, d), lambda i: (i, 0)),
            pl.BlockSpec((_NE, d), lambda i: (0, 0)),
            pl.BlockSpec((1, _NE), lambda i: (0, 0)),
        ],
        out_specs=pl.BlockSpec((block_t, _NE), lambda i: (i, 0)),
        out_shape=jax.ShapeDtypeStruct((n_tok, _NE), jnp.float32),
    )(xf, w, b.reshape(1, _NE))
    return out.reshape(bsz, seq, _NE)


def kernel(x, W, b):
    return _switch_gate(x, W, b, block_t=1024)


# speedup vs baseline: 1.2052x; 1.2052x over previous
import functools

import jax
import jax.numpy as jnp
from jax import lax
from jax.experimental import pallas as pl

_NE = 64
_K = 8
_EPS = 1e-6


def _gate_kernel(x_ref, w_ref, b_ref, out_ref):
    x = x_ref[...]
    w = w_ref[...]
    logits = lax.dot_general(w, x, (((1,), (1,)), ((), ())),
                             preferred_element_type=jnp.float32)
    logits = logits + b_ref[...]
    t = logits.shape[1]
    idx = lax.broadcasted_iota(jnp.int32, (_NE, t), 0)
    m = jnp.max(logits, axis=0, keepdims=True)
    e = jnp.exp(logits - m)
    z = jnp.sum(e, axis=0, keepdims=True)
    tv = jnp.full((1, t), jnp.inf, jnp.float32)
    ti = jnp.full((1, t), -1, jnp.int32)
    for _ in range(_K):
        elig = (logits < tv) | ((logits == tv) & (idx > ti))
        lm = jnp.where(elig, logits, -jnp.inf)
        tv = jnp.max(lm, axis=0, keepdims=True)
        ti = jnp.min(jnp.where(lm == tv, idx, _NE), axis=0, keepdims=True)
    mask = (logits > tv) | ((logits == tv) & (idx <= ti))
    es = jnp.where(mask, e, 0.0)
    s8 = jnp.sum(es, axis=0, keepdims=True)
    out_ref[...] = es / (s8 + _EPS * z)


@functools.partial(jax.jit, static_argnames=("block_t",))
def _switch_gate(x, w, b, block_t=1024):
    bsz, seq, d = x.shape
    n_tok = bsz * seq
    xf = x.reshape(n_tok, d)
    grid = n_tok // block_t
    out_t = pl.pallas_call(
        _gate_kernel,
        grid=(grid,),
        in_specs=[
            pl.BlockSpec((block_t, d), lambda i: (i, 0)),
            pl.BlockSpec((_NE, d), lambda i: (0, 0)),
            pl.BlockSpec((_NE, 1), lambda i: (0, 0)),
        ],
        out_specs=pl.BlockSpec((_NE, block_t), lambda i: (0, i)),
        out_shape=jax.ShapeDtypeStruct((_NE, n_tok), jnp.float32),
    )(xf, w, b.reshape(_NE, 1))
    return out_t.T.reshape(bsz, seq, _NE)


def kernel(x, W, b):
    return _switch_gate(x, W, b, block_t=1024)


# speedup vs baseline: 1.2063x; 1.0009x over previous
import functools

import jax
import jax.numpy as jnp
from jax import lax
from jax.experimental import pallas as pl

_NE = 64
_K = 8
_EPS = 1e-6


def _gate_kernel(x_ref, w_ref, b_ref, out_ref):
    x = x_ref[...]
    w = w_ref[...]
    logits = lax.dot_general(w, x, (((1,), (1,)), ((), ())),
                             preferred_element_type=jnp.float32)
    logits = logits + b_ref[...]
    t = logits.shape[1]
    idx = lax.broadcasted_iota(jnp.int32, (_NE, t), 0)
    m = jnp.max(logits, axis=0, keepdims=True)
    e = jnp.exp(logits - m)
    z = jnp.sum(e, axis=0, keepdims=True)
    tv = jnp.full((1, t), jnp.inf, jnp.float32)
    ti = jnp.full((1, t), -1, jnp.int32)
    for _ in range(_K):
        elig = (logits < tv) | ((logits == tv) & (idx > ti))
        lm = jnp.where(elig, logits, -jnp.inf)
        tv = jnp.max(lm, axis=0, keepdims=True)
        ti = jnp.min(jnp.where(lm == tv, idx, _NE), axis=0, keepdims=True)
    mask = (logits > tv) | ((logits == tv) & (idx <= ti))
    es = jnp.where(mask, e, 0.0)
    s8 = jnp.sum(es, axis=0, keepdims=True)
    out_ref[...] = es / (s8 + _EPS * z)


def _gate_kernel2(xa_ref, xb_ref, w_ref, b_ref, out_ref):
    w = w_ref[...]
    la = lax.dot_general(w, xa_ref[...], (((1,), (1,)), ((), ())),
                         preferred_element_type=jnp.float32)
    lb = lax.dot_general(w, xb_ref[...], (((1,), (1,)), ((), ())),
                         preferred_element_type=jnp.float32)
    logits = jnp.concatenate([la, lb], axis=1) + b_ref[...]
    t = logits.shape[1]
    idx = lax.broadcasted_iota(jnp.int32, (_NE, t), 0)
    m = jnp.max(logits, axis=0, keepdims=True)
    e = jnp.exp(logits - m)
    z = jnp.sum(e, axis=0, keepdims=True)
    tv = jnp.full((1, t), jnp.inf, jnp.float32)
    ti = jnp.full((1, t), -1, jnp.int32)
    for _ in range(_K):
        elig = (logits < tv) | ((logits == tv) & (idx > ti))
        lm = jnp.where(elig, logits, -jnp.inf)
        tv = jnp.max(lm, axis=0, keepdims=True)
        ti = jnp.min(jnp.where(lm == tv, idx, _NE), axis=0, keepdims=True)
    mask = (logits > tv) | ((logits == tv) & (idx <= ti))
    es = jnp.where(mask, e, 0.0)
    s8 = jnp.sum(es, axis=0, keepdims=True)
    out_ref[...] = es / (s8 + _EPS * z)


@functools.partial(jax.jit, static_argnames=("block_t",))
def _switch_gate(x, w, b, block_t=1024):
    bsz, seq, d = x.shape
    n_tok = bsz * seq
    xf = x.reshape(n_tok, d)
    half = block_t // 2
    grid = n_tok // block_t
    out_t = pl.pallas_call(
        _gate_kernel2,
        grid=(grid,),
        in_specs=[
            pl.BlockSpec((half, d), lambda i: (2 * i, 0)),
            pl.BlockSpec((half, d), lambda i: (2 * i + 1, 0)),
            pl.BlockSpec((_NE, d), lambda i: (0, 0)),
            pl.BlockSpec((_NE, 1), lambda i: (0, 0)),
        ],
        out_specs=pl.BlockSpec((_NE, block_t), lambda i: (0, i)),
        out_shape=jax.ShapeDtypeStruct((_NE, n_tok), jnp.float32),
    )(xf, xf, w, b.reshape(_NE, 1))
    return out_t.T.reshape(bsz, seq, _NE)


def kernel(x, W, b):
    return _switch_gate(x, W, b, block_t=1024)
